# baseline (speedup 1.0000x reference)
import functools

import jax
import jax.numpy as jnp
import numpy as np
from jax.experimental import pallas as pl
from jax.experimental.pallas import tpu as pltpu

D = 128
H = 8
B = 16384
RES = 2048
K = 80

ROWS = 1024


def _l2norm_cols(x):
    n = jnp.sqrt(jnp.sum(x * x, axis=0, keepdims=True))
    return x / jnp.maximum(n, 1e-12)


def _tc_body(sg_ref, sl_ref, vf_hbm, w1g_ref, w1l_ref, w2g_ref, w2l_ref,
             wv_ref, wo_ref, b1g_ref, b1l_ref, b2g_ref, b2l_ref,
             bv_ref, bo_ref, tril_ref,
             fused_hbm, fff_ref, f2_ref,
             sp_vmem, sp_sem0, sp_sem1, vf_sem):
    f32 = jnp.float32
    i = pl.program_id(0)
    n = B // ROWS
    rows = pl.ds(i * ROWS, ROWS)

    pltpu.make_async_copy(vf_hbm.at[rows, :], fused_hbm.at[rows, 0:RES],
                          vf_sem).start()

    def dot(w, x):
        return jnp.dot(w, x, preferred_element_type=f32)

    def dot_rt(w, x):
        return jax.lax.dot_general(w, x, (((1,), (1,)), ((), ())),
                                   preferred_element_type=f32)

    h = jax.nn.relu(dot_rt(w1g_ref[...], sg_ref[...]) + b1g_ref[...])
    h = dot(w2g_ref[...], h) + b2g_ref[...]
    sg2T = jax.nn.sigmoid(h[:D]) * h[D:]

    h = jax.nn.relu(dot_rt(w1l_ref[...], sl_ref[...]) + b1l_ref[...])
    h = dot(w2l_ref[...], h) + b2l_ref[...]
    sl2T = jax.nn.sigmoid(h[:D]) * h[D:]

    zT = jnp.concatenate([sg2T, sl2T], axis=1)
    zT = dot(wo_ref[...], dot(wv_ref[...], zT) + bv_ref[...]) + bo_ref[...]
    f1T = _l2norm_cols(zT[:, :ROWS])
    f2T = _l2norm_cols(zT[:, ROWS:])
    fffT = jax.nn.sigmoid(f1T) * f2T

    aT = jax.lax.bitcast_convert_type(jnp.abs(fffT), jnp.int32)
    t = jnp.zeros((1, ROWS), jnp.int32)
    for bit in range(29, -1, -1):
        cand = t | (1 << bit)
        ge = jnp.where(aT >= cand, 1.0, 0.0)
        cnt = jnp.sum(ge, axis=0, keepdims=True)
        t = jnp.where(cnt >= K, cand, t)

    gt = aT > t
    eq = aT == t
    n_gt = jnp.sum(jnp.where(gt, 1.0, 0.0), axis=0, keepdims=True)
    prefix = dot(tril_ref[...], jnp.where(eq, 1.0, 0.0))
    keep = gt | (eq & (prefix <= (K - n_gt)))
    sparseT = jnp.where(keep, fffT, 0.0)
    sparse = sparseT.T
    fff_ref[...] = fffT.T
    f2_ref[...] = f2T.T

    def stage(slot, sem):
        @pl.when(i >= 2)
        def _():
            pltpu.make_async_copy(sp_vmem.at[slot], fused_hbm.at[rows, RES:],
                                  sem).wait()
        sp_vmem[slot] = sparse
        pltpu.make_async_copy(sp_vmem.at[slot], fused_hbm.at[rows, RES:],
                              sem).start()

    parity = jax.lax.rem(i, 2)

    @pl.when(parity == 0)
    def _():
        stage(0, sp_sem0)

    @pl.when(parity == 1)
    def _():
        stage(1, sp_sem1)

    @pl.when(i == n - 1)
    def _():
        last_parity = (n - 1) % 2
        pltpu.make_async_copy(sp_vmem.at[last_parity],
                              fused_hbm.at[rows, RES:],
                              sp_sem0 if last_parity == 0 else sp_sem1).wait()
        pltpu.make_async_copy(sp_vmem.at[1 - last_parity],
                              fused_hbm.at[rows, RES:],
                              sp_sem1 if last_parity == 0 else sp_sem0).wait()
        for _ in range(n):
            pltpu.make_async_copy(vf_hbm.at[rows, :],
                                  fused_hbm.at[rows, 0:RES], vf_sem).wait()


def _run_tc(sg, sl, vf, w1g, w1l, w2g, w2l, wv, wo,
            b1g, b1l, b2g, b2l, bv, bo, tril, *, interpret=False):
    grid = (B // ROWS,)
    row_spec = lambda c: pl.BlockSpec((ROWS, c), lambda i: (i, 0))
    full2 = lambda a, b: pl.BlockSpec((a, b), lambda i: (0, 0))
    return pl.pallas_call(
        _tc_body,
        grid=grid,
        in_specs=[
            row_spec(D), row_spec(D),
            pl.BlockSpec(memory_space=pl.ANY),
            full2(2 * D, D), full2(2 * D, D),
            full2(2 * D, 2 * D), full2(2 * D, 2 * D),
            full2(D, D), full2(D, D),
            full2(2 * D, 1), full2(2 * D, 1),
            full2(2 * D, 1), full2(2 * D, 1),
            full2(D, 1), full2(D, 1),
            full2(D, D),
        ],
        out_specs=[
            pl.BlockSpec(memory_space=pl.ANY),
            row_spec(D), row_spec(D),
        ],
        out_shape=[
            jax.ShapeDtypeStruct((B, RES + D), jnp.float32),
            jax.ShapeDtypeStruct((B, D), jnp.float32),
            jax.ShapeDtypeStruct((B, D), jnp.float32),
        ],
        scratch_shapes=[
            pltpu.VMEM((2, ROWS, D), jnp.float32),
            pltpu.SemaphoreType.DMA,
            pltpu.SemaphoreType.DMA,
            pltpu.SemaphoreType.DMA,
        ],
        compiler_params=pltpu.CompilerParams(
            dimension_semantics=("arbitrary",),
        ),
        interpret=interpret,
    )(sg, sl, vf, w1g, w1l, w2g, w2l, wv, wo,
      b1g, b1l, b2g, b2l, bv, bo, tril)


def kernel(semantic_global, semantic_local, visual_feat, params):
    p = params
    f32 = jnp.float32

    def blockdiag(a, b):
        z = jnp.zeros((D, D), f32)
        return jnp.block([[a, z], [z, b]])

    w1g = jnp.concatenate([p['W_gu1'], p['W_gd1']], axis=0)
    w1l = jnp.concatenate([p['W_lu1'], p['W_ld1']], axis=0)
    w2g = blockdiag(p['W_gu2'], p['W_gd2'])
    w2l = blockdiag(p['W_lu2'], p['W_ld2'])
    wv = p['W_v']
    wo = p['W_o']
    b1g = jnp.concatenate([p['b_gu1'], p['b_gd1']])[:, None]
    b1l = jnp.concatenate([p['b_lu1'], p['b_ld1']])[:, None]
    b2g = jnp.concatenate([p['b_gu2'], p['b_gd2']])[:, None]
    b2l = jnp.concatenate([p['b_lu2'], p['b_ld2']])[:, None]
    bv = p['b_v'][:, None]
    bo = p['b_o'][:, None]
    tril = jnp.tril(jnp.ones((D, D), f32))
    fused, fff, f2 = _run_tc(semantic_global, semantic_local, visual_feat,
                             w1g, w1l, w2g, w2l, wv, wo,
                             b1g, b1l, b2g, b2l, bv, bo, tril)
    return fused, fff, f2


# speedup vs baseline: 23.9167x; 23.9167x over previous
import functools

import jax
import jax.numpy as jnp
import numpy as np
from jax import lax
from jax.experimental import pallas as pl
from jax.experimental.pallas import tpu as pltpu
from jax.experimental.pallas import tpu_sc as plsc

D = 128
H = 8
B = 16384
RES = 2048
K = 80

ROWS = 1024
NW = 32
RPW = B // NW
CH = 64


def _l2norm_cols(x):
    n = jnp.sqrt(jnp.sum(x * x, axis=0, keepdims=True))
    return x / jnp.maximum(n, 1e-12)


def _tc1_body(sg_ref, sl_ref, w1g_ref, w1l_ref, w2g_ref, w2l_ref,
              wv_ref, wo_ref, b1g_ref, b1l_ref, b2g_ref, b2l_ref,
              bv_ref, bo_ref, fff_ref, f2_ref):
    f32 = jnp.float32

    def dot(w, x):
        return jnp.dot(w, x, preferred_element_type=f32)

    def dot_rt(w, x):
        return jax.lax.dot_general(w, x, (((1,), (1,)), ((), ())),
                                   preferred_element_type=f32)

    h = jax.nn.relu(dot_rt(w1g_ref[...], sg_ref[...]) + b1g_ref[...])
    h = dot(w2g_ref[...], h) + b2g_ref[...]
    sg2T = jax.nn.sigmoid(h[:D]) * h[D:]

    h = jax.nn.relu(dot_rt(w1l_ref[...], sl_ref[...]) + b1l_ref[...])
    h = dot(w2l_ref[...], h) + b2l_ref[...]
    sl2T = jax.nn.sigmoid(h[:D]) * h[D:]

    zT = jnp.concatenate([sg2T, sl2T], axis=1)
    zT = dot(wo_ref[...], dot(wv_ref[...], zT) + bv_ref[...]) + bo_ref[...]
    f1T = _l2norm_cols(zT[:, :ROWS])
    f2T = _l2norm_cols(zT[:, ROWS:])
    fffT = jax.nn.sigmoid(f1T) * f2T
    fff_ref[...] = fffT.T
    f2_ref[...] = f2T.T


def _asm_body(vf_ref, sp_ref, fused_ref):
    fused_ref[:, :RES] = vf_ref[...]
    fused_ref[:, RES:] = sp_ref[...]


def _rev(x):
    return lax.rev(x, (0,))


def _sort(x):
    return plsc.sort_key_val(x, x)[0]


def _mm(a, b):
    return jnp.minimum(a, b), jnp.maximum(a, b)


def _m2(a, b):
    lo, hi = _mm(a, _rev(b))
    return _sort(lo), _sort(hi)


def _bm2(x0, x1):
    lo, hi = _mm(x0, x1)
    return _sort(lo), _sort(hi)


def _m4(A, Bq):
    x0, y0 = _mm(A[0], _rev(Bq[1]))
    x1, y1 = _mm(A[1], _rev(Bq[0]))
    return list(_bm2(x0, x1)) + list(_bm2(y0, y1))


def _bm4(x):
    l0, h0 = _mm(x[0], x[2])
    l1, h1 = _mm(x[1], x[3])
    return list(_bm2(l0, l1)) + list(_bm2(h0, h1))


def _m8(A, Bq):
    rB = [_rev(Bq[3]), _rev(Bq[2]), _rev(Bq[1]), _rev(Bq[0])]
    X = [jnp.minimum(A[k], rB[k]) for k in range(4)]
    Y = [jnp.maximum(A[k], rB[k]) for k in range(4)]
    return _bm4(X) + _bm4(Y)


def _row_threshold(a):
    s = [_sort(x) for x in a]
    p01 = _m2(s[0], s[1])
    p23 = _m2(s[2], s[3])
    p45 = _m2(s[4], s[5])
    p67 = _m2(s[6], s[7])
    q0 = _m4(list(p01), list(p23))
    q1 = _m4(list(p45), list(p67))
    o = _m8(q0, q1)
    return jnp.min(o[3])


def _make_sc_topk():
    mesh = plsc.VectorSubcoreMesh(core_axis_name="c", subcore_axis_name="s")

    @functools.partial(
        pl.kernel, mesh=mesh,
        out_type=jax.ShapeDtypeStruct((B, D), jnp.float32),
        scratch_types=[
            pltpu.VMEM((CH, D), jnp.float32),
            pltpu.VMEM((CH, D), jnp.float32),
        ],
        compiler_params=pltpu.CompilerParams(needs_layout_passes=False),
    )
    def sc_topk(fff_hbm, sparse_hbm, inbuf, outbuf):
        wid = lax.axis_index("s") * 2 + lax.axis_index("c")
        base = wid * RPW

        def do_row(j):
            v = [inbuf[j, pl.ds(k * 16, 16)] for k in range(8)]
            t = _row_threshold([jnp.abs(x) for x in v])
            for k in range(8):
                outbuf[j, pl.ds(k * 16, 16)] = jnp.where(
                    jnp.abs(v[k]) >= t, v[k], 0.0)

        def chunk_body(c, carry):
            r0 = base + c * CH
            pltpu.sync_copy(fff_hbm.at[pl.ds(r0, CH)], inbuf)

            def row_body(jj, carry2):
                do_row(2 * jj)
                do_row(2 * jj + 1)
                return carry2

            lax.fori_loop(0, CH // 2, row_body, 0)
            pltpu.sync_copy(outbuf, sparse_hbm.at[pl.ds(r0, CH)])
            return carry

        lax.fori_loop(0, RPW // CH, chunk_body, 0)

    return sc_topk


def _run_tc1(sg, sl, w1g, w1l, w2g, w2l, wv, wo,
             b1g, b1l, b2g, b2l, bv, bo, *, interpret=False):
    grid = (B // ROWS,)
    row_spec = lambda c: pl.BlockSpec((ROWS, c), lambda i: (i, 0))
    full2 = lambda a, b: pl.BlockSpec((a, b), lambda i: (0, 0))
    return pl.pallas_call(
        _tc1_body,
        grid=grid,
        in_specs=[
            row_spec(D), row_spec(D),
            full2(2 * D, D), full2(2 * D, D),
            full2(2 * D, 2 * D), full2(2 * D, 2 * D),
            full2(D, D), full2(D, D),
            full2(2 * D, 1), full2(2 * D, 1),
            full2(2 * D, 1), full2(2 * D, 1),
            full2(D, 1), full2(D, 1),
        ],
        out_specs=[row_spec(D), row_spec(D)],
        out_shape=[
            jax.ShapeDtypeStruct((B, D), jnp.float32),
            jax.ShapeDtypeStruct((B, D), jnp.float32),
        ],
        compiler_params=pltpu.CompilerParams(
            dimension_semantics=("arbitrary",),
        ),
        interpret=interpret,
    )(sg, sl, w1g, w1l, w2g, w2l, wv, wo, b1g, b1l, b2g, b2l, bv, bo)


def _run_asm(vf, sp, *, interpret=False):
    grid = (B // ROWS,)
    row_spec = lambda c: pl.BlockSpec((ROWS, c), lambda i: (i, 0))
    return pl.pallas_call(
        _asm_body,
        grid=grid,
        in_specs=[row_spec(RES), row_spec(D)],
        out_specs=row_spec(RES + D),
        out_shape=jax.ShapeDtypeStruct((B, RES + D), jnp.float32),
        compiler_params=pltpu.CompilerParams(
            dimension_semantics=("arbitrary",),
        ),
        interpret=interpret,
    )(vf, sp)


def kernel(semantic_global, semantic_local, visual_feat, params):
    p = params
    f32 = jnp.float32

    def blockdiag(a, b):
        z = jnp.zeros((D, D), f32)
        return jnp.block([[a, z], [z, b]])

    w1g = jnp.concatenate([p['W_gu1'], p['W_gd1']], axis=0)
    w1l = jnp.concatenate([p['W_lu1'], p['W_ld1']], axis=0)
    w2g = blockdiag(p['W_gu2'], p['W_gd2'])
    w2l = blockdiag(p['W_lu2'], p['W_ld2'])
    wv = p['W_v']
    wo = p['W_o']
    b1g = jnp.concatenate([p['b_gu1'], p['b_gd1']])[:, None]
    b1l = jnp.concatenate([p['b_lu1'], p['b_ld1']])[:, None]
    b2g = jnp.concatenate([p['b_gu2'], p['b_gd2']])[:, None]
    b2l = jnp.concatenate([p['b_lu2'], p['b_ld2']])[:, None]
    bv = p['b_v'][:, None]
    bo = p['b_o'][:, None]
    fff, f2 = _run_tc1(semantic_global, semantic_local,
                       w1g, w1l, w2g, w2l, wv, wo,
                       b1g, b1l, b2g, b2l, bv, bo)
    sparse = _make_sc_topk()(fff)
    fused = _run_asm(visual_feat, sparse)
    return fused, fff, f2


# speedup vs baseline: 24.8955x; 1.0409x over previous
import functools

import jax
import jax.numpy as jnp
import numpy as np
from jax import lax
from jax.experimental import pallas as pl
from jax.experimental.pallas import tpu as pltpu
from jax.experimental.pallas import tpu_sc as plsc

D = 128
H = 8
B = 16384
RES = 2048
K = 80

ROWS = 1024
NW = 32
RPW = B // NW
CH = 64


def _l2norm_cols(x):
    n = jnp.sqrt(jnp.sum(x * x, axis=0, keepdims=True))
    return x / jnp.maximum(n, 1e-12)


def _tc1_body(sg_ref, sl_ref, vf_ref, w1g_ref, w1l_ref, w2g_ref, w2l_ref,
              wv_ref, wo_ref, b1g_ref, b1l_ref, b2g_ref, b2l_ref,
              bv_ref, bo_ref, fused_ref, fff_ref, f2_ref):
    f32 = jnp.float32

    def dot(w, x):
        return jnp.dot(w, x, preferred_element_type=f32)

    def dot_rt(w, x):
        return jax.lax.dot_general(w, x, (((1,), (1,)), ((), ())),
                                   preferred_element_type=f32)

    h = jax.nn.relu(dot_rt(w1g_ref[...], sg_ref[...]) + b1g_ref[...])
    h = dot(w2g_ref[...], h) + b2g_ref[...]
    sg2T = jax.nn.sigmoid(h[:D]) * h[D:]

    h = jax.nn.relu(dot_rt(w1l_ref[...], sl_ref[...]) + b1l_ref[...])
    h = dot(w2l_ref[...], h) + b2l_ref[...]
    sl2T = jax.nn.sigmoid(h[:D]) * h[D:]

    zT = jnp.concatenate([sg2T, sl2T], axis=1)
    zT = dot(wo_ref[...], dot(wv_ref[...], zT) + bv_ref[...]) + bo_ref[...]
    f1T = _l2norm_cols(zT[:, :ROWS])
    f2T = _l2norm_cols(zT[:, ROWS:])
    fffT = jax.nn.sigmoid(f1T) * f2T
    fused_ref[:, :RES] = vf_ref[...]
    fused_ref[:, RES:] = jnp.zeros((ROWS, D), f32)
    fff_ref[...] = fffT.T
    f2_ref[...] = f2T.T


def _patch_body(sp_ref, fused_in_any, fused_strip_ref):
    del fused_in_any
    fused_strip_ref[...] = sp_ref[...]


def _rev(x):
    return lax.rev(x, (0,))


def _sort(x):
    return plsc.sort_key_val(x, x)[0]


def _mm(a, b):
    return jnp.minimum(a, b), jnp.maximum(a, b)


def _m2(a, b):
    lo, hi = _mm(a, _rev(b))
    return _sort(lo), _sort(hi)


def _bm2(x0, x1):
    lo, hi = _mm(x0, x1)
    return _sort(lo), _sort(hi)


def _m4(A, Bq):
    x0, y0 = _mm(A[0], _rev(Bq[1]))
    x1, y1 = _mm(A[1], _rev(Bq[0]))
    return list(_bm2(x0, x1)) + list(_bm2(y0, y1))


def _bm4(x):
    l0, h0 = _mm(x[0], x[2])
    l1, h1 = _mm(x[1], x[3])
    return list(_bm2(l0, l1)) + list(_bm2(h0, h1))


def _m8(A, Bq):
    rB = [_rev(Bq[3]), _rev(Bq[2]), _rev(Bq[1]), _rev(Bq[0])]
    X = [jnp.minimum(A[k], rB[k]) for k in range(4)]
    Y = [jnp.maximum(A[k], rB[k]) for k in range(4)]
    return _bm4(X) + _bm4(Y)


def _row_threshold(a):
    s = [_sort(x) for x in a]
    p01 = _m2(s[0], s[1])
    p23 = _m2(s[2], s[3])
    p45 = _m2(s[4], s[5])
    p67 = _m2(s[6], s[7])
    q0 = _m4(list(p01), list(p23))
    q1 = _m4(list(p45), list(p67))
    o = _m8(q0, q1)
    return jnp.min(o[3])


def _make_sc_topk():
    mesh = plsc.VectorSubcoreMesh(core_axis_name="c", subcore_axis_name="s")

    @functools.partial(
        pl.kernel, mesh=mesh,
        out_type=jax.ShapeDtypeStruct((B, D), jnp.float32),
        scratch_types=[
            pltpu.VMEM((CH, D), jnp.float32),
            pltpu.VMEM((CH, D), jnp.float32),
        ],
        compiler_params=pltpu.CompilerParams(needs_layout_passes=False),
    )
    def sc_topk(fff_hbm, sparse_hbm, inbuf, outbuf):
        wid = lax.axis_index("s") * 2 + lax.axis_index("c")
        base = wid * RPW

        def do_row(j):
            v = [inbuf[j, pl.ds(k * 16, 16)] for k in range(8)]
            t = _row_threshold([jnp.abs(x) for x in v])
            for k in range(8):
                outbuf[j, pl.ds(k * 16, 16)] = jnp.where(
                    jnp.abs(v[k]) >= t, v[k], 0.0)

        def chunk_body(c, carry):
            r0 = base + c * CH
            pltpu.sync_copy(fff_hbm.at[pl.ds(r0, CH)], inbuf)

            def row_body(jj, carry2):
                do_row(2 * jj)
                do_row(2 * jj + 1)
                return carry2

            lax.fori_loop(0, CH // 2, row_body, 0)
            pltpu.sync_copy(outbuf, sparse_hbm.at[pl.ds(r0, CH)])
            return carry

        lax.fori_loop(0, RPW // CH, chunk_body, 0)

    return sc_topk


def _run_tc1(sg, sl, vf, w1g, w1l, w2g, w2l, wv, wo,
             b1g, b1l, b2g, b2l, bv, bo, *, interpret=False):
    grid = (B // ROWS,)
    row_spec = lambda c: pl.BlockSpec((ROWS, c), lambda i: (i, 0))
    full2 = lambda a, b: pl.BlockSpec((a, b), lambda i: (0, 0))
    return pl.pallas_call(
        _tc1_body,
        grid=grid,
        in_specs=[
            row_spec(D), row_spec(D), row_spec(RES),
            full2(2 * D, D), full2(2 * D, D),
            full2(2 * D, 2 * D), full2(2 * D, 2 * D),
            full2(D, D), full2(D, D),
            full2(2 * D, 1), full2(2 * D, 1),
            full2(2 * D, 1), full2(2 * D, 1),
            full2(D, 1), full2(D, 1),
        ],
        out_specs=[row_spec(RES + D), row_spec(D), row_spec(D)],
        out_shape=[
            jax.ShapeDtypeStruct((B, RES + D), jnp.float32),
            jax.ShapeDtypeStruct((B, D), jnp.float32),
            jax.ShapeDtypeStruct((B, D), jnp.float32),
        ],
        compiler_params=pltpu.CompilerParams(
            dimension_semantics=("arbitrary",),
        ),
        interpret=interpret,
    )(sg, sl, vf, w1g, w1l, w2g, w2l, wv, wo, b1g, b1l, b2g, b2l, bv, bo)


def _run_patch(sp, fused0, *, interpret=False):
    grid = (B // ROWS,)
    return pl.pallas_call(
        _patch_body,
        grid=grid,
        in_specs=[
            pl.BlockSpec((ROWS, D), lambda i: (i, 0)),
            pl.BlockSpec(memory_space=pl.ANY),
        ],
        out_specs=pl.BlockSpec((ROWS, D), lambda i: (i, RES // D)),
        out_shape=jax.ShapeDtypeStruct((B, RES + D), jnp.float32),
        input_output_aliases={1: 0},
        compiler_params=pltpu.CompilerParams(
            dimension_semantics=("arbitrary",),
        ),
        interpret=interpret,
    )(sp, fused0)


def kernel(semantic_global, semantic_local, visual_feat, params):
    p = params
    f32 = jnp.float32

    def blockdiag(a, b):
        z = jnp.zeros((D, D), f32)
        return jnp.block([[a, z], [z, b]])

    w1g = jnp.concatenate([p['W_gu1'], p['W_gd1']], axis=0)
    w1l = jnp.concatenate([p['W_lu1'], p['W_ld1']], axis=0)
    w2g = blockdiag(p['W_gu2'], p['W_gd2'])
    w2l = blockdiag(p['W_lu2'], p['W_ld2'])
    wv = p['W_v']
    wo = p['W_o']
    b1g = jnp.concatenate([p['b_gu1'], p['b_gd1']])[:, None]
    b1l = jnp.concatenate([p['b_lu1'], p['b_ld1']])[:, None]
    b2g = jnp.concatenate([p['b_gu2'], p['b_gd2']])[:, None]
    b2l = jnp.concatenate([p['b_lu2'], p['b_ld2']])[:, None]
    bv = p['b_v'][:, None]
    bo = p['b_o'][:, None]
    fused0, fff, f2 = _run_tc1(semantic_global, semantic_local, visual_feat,
                               w1g, w1l, w2g, w2l, wv, wo,
                               b1g, b1l, b2g, b2l, bv, bo)
    sparse = _make_sc_topk()(fff)
    fused = _run_patch(sparse, fused0)
    return fused, fff, f2


# speedup vs baseline: 24.9525x; 1.0023x over previous
import functools

import jax
import jax.numpy as jnp
import numpy as np
from jax import lax
from jax.experimental import pallas as pl
from jax.experimental.pallas import tpu as pltpu
from jax.experimental.pallas import tpu_sc as plsc

D = 128
H = 8
B = 16384
RES = 2048
K = 80

ROWS = 1024
HB = B // 2
NBLK_H = HB // ROWS
NW = 32
CH = 64


def _l2norm_cols(x):
    n = jnp.sqrt(jnp.sum(x * x, axis=0, keepdims=True))
    return x / jnp.maximum(n, 1e-12)


def _tc1_body(sg_ref, sl_ref, vf_ref, w1g_ref, w1l_ref, w2g_ref, w2l_ref,
              wv_ref, wo_ref, b1g_ref, b1l_ref, b2g_ref, b2l_ref,
              bv_ref, bo_ref, fused_ref, fff_ref, f2_ref):
    f32 = jnp.float32

    def dot(w, x):
        return jnp.dot(w, x, preferred_element_type=f32)

    def dot_rt(w, x):
        return jax.lax.dot_general(w, x, (((1,), (1,)), ((), ())),
                                   preferred_element_type=f32)

    h = jax.nn.relu(dot_rt(w1g_ref[...], sg_ref[...]) + b1g_ref[...])
    h = dot(w2g_ref[...], h) + b2g_ref[...]
    sg2T = jax.nn.sigmoid(h[:D]) * h[D:]

    h = jax.nn.relu(dot_rt(w1l_ref[...], sl_ref[...]) + b1l_ref[...])
    h = dot(w2l_ref[...], h) + b2l_ref[...]
    sl2T = jax.nn.sigmoid(h[:D]) * h[D:]

    zT = jnp.concatenate([sg2T, sl2T], axis=1)
    zT = dot(wo_ref[...], dot(wv_ref[...], zT) + bv_ref[...]) + bo_ref[...]
    f1T = _l2norm_cols(zT[:, :ROWS])
    f2T = _l2norm_cols(zT[:, ROWS:])
    fffT = jax.nn.sigmoid(f1T) * f2T
    fused_ref[:, :RES] = vf_ref[...]
    fused_ref[:, RES:] = jnp.zeros((ROWS, D), f32)
    fff_ref[...] = fffT.T
    f2_ref[...] = f2T.T


def _patch_body(sp_ref, fused_in_any, fused_strip_ref):
    del fused_in_any
    fused_strip_ref[...] = sp_ref[...]


def _rev(x):
    return lax.rev(x, (0,))


def _sort(x):
    return plsc.sort_key_val(x, x)[0]


def _mm(a, b):
    return jnp.minimum(a, b), jnp.maximum(a, b)


def _m2(a, b):
    lo, hi = _mm(a, _rev(b))
    return _sort(lo), _sort(hi)


def _bm2(x0, x1):
    lo, hi = _mm(x0, x1)
    return _sort(lo), _sort(hi)


def _m4(A, Bq):
    x0, y0 = _mm(A[0], _rev(Bq[1]))
    x1, y1 = _mm(A[1], _rev(Bq[0]))
    return list(_bm2(x0, x1)) + list(_bm2(y0, y1))


def _bm4(x):
    l0, h0 = _mm(x[0], x[2])
    l1, h1 = _mm(x[1], x[3])
    return list(_bm2(l0, l1)) + list(_bm2(h0, h1))


def _m8(A, Bq):
    rB = [_rev(Bq[3]), _rev(Bq[2]), _rev(Bq[1]), _rev(Bq[0])]
    X = [jnp.minimum(A[k], rB[k]) for k in range(4)]
    Y = [jnp.maximum(A[k], rB[k]) for k in range(4)]
    return _bm4(X) + _bm4(Y)


def _row_threshold(a):
    s = [_sort(x) for x in a]
    p01 = _m2(s[0], s[1])
    p23 = _m2(s[2], s[3])
    p45 = _m2(s[4], s[5])
    p67 = _m2(s[6], s[7])
    q0 = _m4(list(p01), list(p23))
    q1 = _m4(list(p45), list(p67))
    o = _m8(q0, q1)
    return jnp.min(o[3])


def _make_sc_topk(nrows):
    mesh = plsc.VectorSubcoreMesh(core_axis_name="c", subcore_axis_name="s")
    rpw = nrows // NW

    @functools.partial(
        pl.kernel, mesh=mesh,
        out_type=jax.ShapeDtypeStruct((nrows, D), jnp.float32),
        scratch_types=[
            pltpu.VMEM((CH, D), jnp.float32),
            pltpu.VMEM((CH, D), jnp.float32),
        ],
        compiler_params=pltpu.CompilerParams(needs_layout_passes=False),
    )
    def sc_topk(fff_hbm, sparse_hbm, inbuf, outbuf):
        wid = lax.axis_index("s") * 2 + lax.axis_index("c")
        base = wid * rpw

        def do_row(j):
            v = [inbuf[j, pl.ds(k * 16, 16)] for k in range(8)]
            t = _row_threshold([jnp.abs(x) for x in v])
            for k in range(8):
                outbuf[j, pl.ds(k * 16, 16)] = jnp.where(
                    jnp.abs(v[k]) >= t, v[k], 0.0)

        def chunk_body(c, carry):
            r0 = base + c * CH
            pltpu.sync_copy(fff_hbm.at[pl.ds(r0, CH)], inbuf)

            def row_body(jj, carry2):
                do_row(2 * jj)
                do_row(2 * jj + 1)
                return carry2

            lax.fori_loop(0, CH // 2, row_body, 0)
            pltpu.sync_copy(outbuf, sparse_hbm.at[pl.ds(r0, CH)])
            return carry

        lax.fori_loop(0, rpw // CH, chunk_body, 0)

    return sc_topk


def _run_tc1(h, sg, sl, vf, w1g, w1l, w2g, w2l, wv, wo,
             b1g, b1l, b2g, b2l, bv, bo, fused_prev=None, *, interpret=False):
    grid = (NBLK_H,)
    off_spec = lambda c: pl.BlockSpec((ROWS, c), lambda i: (i + h * NBLK_H, 0))
    half_spec = lambda c: pl.BlockSpec((ROWS, c), lambda i: (i, 0))
    full2 = lambda a, b: pl.BlockSpec((a, b), lambda i: (0, 0))
    in_specs = [
        off_spec(D), off_spec(D), off_spec(RES),
        full2(2 * D, D), full2(2 * D, D),
        full2(2 * D, 2 * D), full2(2 * D, 2 * D),
        full2(D, D), full2(D, D),
        full2(2 * D, 1), full2(2 * D, 1),
        full2(2 * D, 1), full2(2 * D, 1),
        full2(D, 1), full2(D, 1),
    ]
    args = [sg, sl, vf, w1g, w1l, w2g, w2l, wv, wo,
            b1g, b1l, b2g, b2l, bv, bo]
    body = _tc1_body
    aliases = {}
    if fused_prev is not None:
        in_specs = [pl.BlockSpec(memory_space=pl.ANY)] + in_specs
        args = [fused_prev] + args
        aliases = {0: 0}

        def body(prev_any, *refs):
            del prev_any
            _tc1_body(*refs)

    return pl.pallas_call(
        body,
        grid=grid,
        in_specs=in_specs,
        out_specs=[off_spec(RES + D), half_spec(D), half_spec(D)],
        out_shape=[
            jax.ShapeDtypeStruct((B, RES + D), jnp.float32),
            jax.ShapeDtypeStruct((HB, D), jnp.float32),
            jax.ShapeDtypeStruct((HB, D), jnp.float32),
        ],
        input_output_aliases=aliases,
        compiler_params=pltpu.CompilerParams(
            dimension_semantics=("arbitrary",),
        ),
        interpret=interpret,
    )(*args)


def _run_patch(h, sp, fused0, *, interpret=False):
    grid = (NBLK_H,)
    return pl.pallas_call(
        _patch_body,
        grid=grid,
        in_specs=[
            pl.BlockSpec((ROWS, D), lambda i: (i, 0)),
            pl.BlockSpec(memory_space=pl.ANY),
        ],
        out_specs=pl.BlockSpec((ROWS, D),
                               lambda i: (i + h * NBLK_H, RES // D)),
        out_shape=jax.ShapeDtypeStruct((B, RES + D), jnp.float32),
        input_output_aliases={1: 0},
        compiler_params=pltpu.CompilerParams(
            dimension_semantics=("arbitrary",),
        ),
        interpret=interpret,
    )(sp, fused0)


def kernel(semantic_global, semantic_local, visual_feat, params):
    p = params
    f32 = jnp.float32

    def blockdiag(a, b):
        z = jnp.zeros((D, D), f32)
        return jnp.block([[a, z], [z, b]])

    w1g = jnp.concatenate([p['W_gu1'], p['W_gd1']], axis=0)
    w1l = jnp.concatenate([p['W_lu1'], p['W_ld1']], axis=0)
    w2g = blockdiag(p['W_gu2'], p['W_gd2'])
    w2l = blockdiag(p['W_lu2'], p['W_ld2'])
    wv = p['W_v']
    wo = p['W_o']
    b1g = jnp.concatenate([p['b_gu1'], p['b_gd1']])[:, None]
    b1l = jnp.concatenate([p['b_lu1'], p['b_ld1']])[:, None]
    b2g = jnp.concatenate([p['b_gu2'], p['b_gd2']])[:, None]
    b2l = jnp.concatenate([p['b_lu2'], p['b_ld2']])[:, None]
    bv = p['b_v'][:, None]
    bo = p['b_o'][:, None]
    w = (w1g, w1l, w2g, w2l, wv, wo, b1g, b1l, b2g, b2l, bv, bo)
    sc = _make_sc_topk(HB)
    fused_a, fff_a, f2_a = _run_tc1(0, semantic_global, semantic_local,
                                    visual_feat, *w)
    fused_b, fff_b, f2_b = _run_tc1(1, semantic_global, semantic_local,
                                    visual_feat, *w, fused_prev=fused_a)
    sp_a = sc(fff_a)
    sp_b = sc(fff_b)
    fused1 = _run_patch(0, sp_a, fused_b)
    fused = _run_patch(1, sp_b, fused1)
    fff = jnp.concatenate([fff_a, fff_b], axis=0)
    f2 = jnp.concatenate([f2_a, f2_b], axis=0)
    return fused, fff, f2
